# Initial kernel scaffold; baseline (speedup 1.0000x reference)
import jax
import jax.numpy as jnp
from jax.experimental import pallas as pl


def kernel(x):
    raise NotImplementedError("write your pallas kernel here")


# speedup vs baseline: 26.8665x; 26.8665x over previous
import functools

import jax
import jax.numpy as jnp
from jax.experimental import pallas as pl
from jax.experimental.pallas import tpu as pltpu

_K1, _K2 = 8, 8
_S1, _S2 = 4, 4
_N = 64
_H = _S1 * (_N - 1) + _K1
_W = _S2 * (_N - 1) + _K2


def _fold_kernel(x_ref, e_ref, o_ref):
    e = e_ref[...]
    for r in range(_S1):
        top = x_ref[0, r]
        bot = x_ref[0, r + _S1]
        wt = jnp.concatenate([top[j] for j in range(_K2)], axis=-1)
        wb = jnp.concatenate([bot[j] for j in range(_K2)], axis=-1)
        zrow = jnp.zeros((1, _K2 * _N), dtype=wt.dtype)
        v = (jnp.concatenate([wt, zrow], axis=0)
             + jnp.concatenate([zrow, wb], axis=0))
        o_ref[0, :, r, :] = jnp.dot(v, e, preferred_element_type=jnp.float32)


@functools.partial(jax.jit, static_argnames=())
def kernel(x):
    b, c, k1, k2, l = x.shape
    n = _N
    xv = x.reshape(b * c, k1, k2, n, n)

    rows = jax.lax.broadcasted_iota(jnp.int32, (k2 * n, _W), 0)
    cols = jax.lax.broadcasted_iota(jnp.int32, (k2 * n, _W), 1)
    dj = rows // n
    pj = rows % n
    e = (cols == _S2 * pj + dj).astype(jnp.float32)

    out = pl.pallas_call(
        _fold_kernel,
        grid=(b * c,),
        in_specs=[
            pl.BlockSpec((1, k1, k2, n, n), lambda i: (i, 0, 0, 0, 0)),
            pl.BlockSpec((k2 * n, _W), lambda i: (0, 0)),
        ],
        out_specs=pl.BlockSpec((1, _H // _S1 + 0, _S1, _W),
                               lambda i: (i, 0, 0, 0)),
        out_shape=jax.ShapeDtypeStruct((b * c, _H // _S1, _S1, _W),
                                       jnp.float32),
        compiler_params=pltpu.CompilerParams(
            dimension_semantics=(pltpu.GridDimensionSemantics.PARALLEL,),
        ),
    )(xv, e)
    return out.reshape(b, c, _H, _W)


# speedup vs baseline: 65.5413x; 2.4395x over previous
import functools

import jax
import jax.numpy as jnp
from jax.experimental import pallas as pl
from jax.experimental.pallas import tpu as pltpu

_K1, _K2 = 8, 8
_S1, _S2 = 4, 4
_N = 64
_H = _S1 * (_N - 1) + _K1
_W = _S2 * (_N - 1) + _K2


def _fold_kernel(x_ref, e_ref, o_ref):
    e = e_ref[...]
    slab = x_ref[0, 0].reshape(_K1 * _K2, _N * _N)
    b3 = jnp.stack([slab[:, _N * p:_N * (p + 1)] for p in range(_N)], axis=0)
    rows = []
    for r in range(_S1):
        top = b3[:, 8 * r:8 * r + 8, :]
        bot = b3[:, 8 * (r + _S1):8 * (r + _S1) + 8, :]
        wt = jnp.concatenate([top[:, j, :] for j in range(_K2)], axis=-1)
        wb = jnp.concatenate([bot[:, j, :] for j in range(_K2)], axis=-1)
        zrow = jnp.zeros((1, _K2 * _N), dtype=wt.dtype)
        v = (jnp.concatenate([wt, zrow], axis=0)
             + jnp.concatenate([zrow, wb], axis=0))
        rows.append(jnp.dot(v, e, preferred_element_type=jnp.float32))
    z = jnp.stack(rows, axis=1)
    o_ref[0, 0] = z.reshape(_H, _W)


@functools.partial(jax.jit, static_argnames=())
def kernel(x):
    b, c, k1, k2, l = x.shape
    n = _N

    rows = jax.lax.broadcasted_iota(jnp.int32, (k2 * n, _W), 0)
    cols = jax.lax.broadcasted_iota(jnp.int32, (k2 * n, _W), 1)
    dj = rows // n
    pj = rows % n
    e = (cols == _S2 * pj + dj).astype(jnp.float32)

    out = pl.pallas_call(
        _fold_kernel,
        grid=(b * c,),
        in_specs=[
            pl.BlockSpec((1, 1, k1, k2, l), lambda i: (i // 32, i % 32, 0, 0, 0)),
            pl.BlockSpec((k2 * n, _W), lambda i: (0, 0)),
        ],
        out_specs=pl.BlockSpec((1, 1, _H, _W), lambda i: (i // 32, i % 32, 0, 0)),
        out_shape=jax.ShapeDtypeStruct((b, c, _H, _W), jnp.float32),
        compiler_params=pltpu.CompilerParams(
            dimension_semantics=(pltpu.GridDimensionSemantics.PARALLEL,),
        ),
    )(x, e)
    return out


# speedup vs baseline: 66.6317x; 1.0166x over previous
import functools

import jax
import jax.numpy as jnp
from jax.experimental import pallas as pl
from jax.experimental.pallas import tpu as pltpu

_K1, _K2 = 8, 8
_S1, _S2 = 4, 4
_N = 64
_H = _S1 * (_N - 1) + _K1
_W = _S2 * (_N - 1) + _K2


def _fold_kernel(x_ref, e_ref, o_ref):
    e = e_ref[...]

    def deinterleave(t8):
        bp = jnp.stack([t8[:, _N * p:_N * (p + 1)] for p in range(_N)],
                       axis=0)
        return jnp.concatenate([bp[:, j, :] for j in range(_K2)], axis=-1)

    for cc in range(2):
        slab = x_ref[0, cc].reshape(_K1 * _K2, _N * _N)
        rows = []
        for r in range(_S1):
            wt = deinterleave(slab[8 * r:8 * r + 8, :])
            wb = deinterleave(slab[8 * (r + _S1):8 * (r + _S1) + 8, :])
            zrow = jnp.zeros((1, _K2 * _N), dtype=wt.dtype)
            v = (jnp.concatenate([wt, zrow], axis=0)
                 + jnp.concatenate([zrow, wb], axis=0))
            rows.append(jnp.dot(v, e, preferred_element_type=jnp.float32))
        z = jnp.stack(rows, axis=1)
        o_ref[0, cc] = z.reshape(_H, _W)


@functools.partial(jax.jit, static_argnames=())
def kernel(x):
    b, c, k1, k2, l = x.shape
    n = _N

    rows = jax.lax.broadcasted_iota(jnp.int32, (k2 * n, _W), 0)
    cols = jax.lax.broadcasted_iota(jnp.int32, (k2 * n, _W), 1)
    dj = rows // n
    pj = rows % n
    e = (cols == _S2 * pj + dj).astype(jnp.float32)

    out = pl.pallas_call(
        _fold_kernel,
        grid=(b * c // 2,),
        in_specs=[
            pl.BlockSpec((1, 2, k1, k2, l), lambda i: (i // 16, i % 16, 0, 0, 0)),
            pl.BlockSpec((k2 * n, _W), lambda i: (0, 0)),
        ],
        out_specs=pl.BlockSpec((1, 2, _H, _W), lambda i: (i // 16, i % 16, 0, 0)),
        out_shape=jax.ShapeDtypeStruct((b, c, _H, _W), jnp.float32),
        compiler_params=pltpu.CompilerParams(
            dimension_semantics=(pltpu.GridDimensionSemantics.PARALLEL,),
        ),
    )(x, e)
    return out


# speedup vs baseline: 66.9053x; 1.0041x over previous
import functools

import jax
import jax.numpy as jnp
from jax.experimental import pallas as pl
from jax.experimental.pallas import tpu as pltpu

_K1, _K2 = 8, 8
_S1, _S2 = 4, 4
_N = 64
_H = _S1 * (_N - 1) + _K1
_W = _S2 * (_N - 1) + _K2


def _fold_kernel(x_ref, e_ref, o_ref):
    e = e_ref[...]

    def deinterleave(t8):
        bp = jnp.stack([t8[:, _N * p:_N * (p + 1)] for p in range(_N)],
                       axis=0)
        return jnp.concatenate([bp[:, j, :] for j in range(_K2)], axis=-1)

    for cc in range(2):
        slab = x_ref[0, cc].reshape(_K1 * _K2, _N * _N)
        rows = []
        for r in range(_S1):
            wt = deinterleave(slab[8 * r:8 * r + 8, :])
            wb = deinterleave(slab[8 * (r + _S1):8 * (r + _S1) + 8, :])
            zrow = jnp.zeros((1, _K2 * _N), dtype=wt.dtype)
            v = (jnp.concatenate([wt, zrow], axis=0)
                 + jnp.concatenate([zrow, wb], axis=0))
            rows.append(jnp.dot(v, e, preferred_element_type=jnp.float32))
        z = jnp.stack(rows, axis=1)
        o_ref[0, cc] = z.reshape(_H, _W)


@functools.partial(jax.jit, static_argnames=())
def kernel(x):
    b, c, k1, k2, l = x.shape
    n = _N

    rows = jax.lax.broadcasted_iota(jnp.int32, (k2 * n, _W), 0)
    cols = jax.lax.broadcasted_iota(jnp.int32, (k2 * n, _W), 1)
    dj = rows // n
    pj = rows % n
    e = (cols == _S2 * pj + dj).astype(jnp.float32)

    out = pl.pallas_call(
        _fold_kernel,
        grid=(b * c // 2,),
        in_specs=[
            pl.BlockSpec((1, 2, k1, k2, l), lambda i: (i // 16, i % 16, 0, 0, 0)),
            pl.BlockSpec((k2 * n, _W), lambda i: (0, 0)),
        ],
        out_specs=pl.BlockSpec((1, 2, _H, _W), lambda i: (i // 16, i % 16, 0, 0)),
        out_shape=jax.ShapeDtypeStruct((b, c, _H, _W), jnp.float32),
        compiler_params=pltpu.CompilerParams(
            dimension_semantics=(pltpu.GridDimensionSemantics.PARALLEL,),
            vmem_limit_bytes=50 * 1024 * 1024,
        ),
    )(x, e)
    return out
